# Initial kernel scaffold; baseline (speedup 1.0000x reference)
import jax
import jax.numpy as jnp
from jax.experimental import pallas as pl


def kernel(x, batch_id_all, batch_id, W1, a_src1, a_dst1, b1, W2, a_src2, a_dst2, b2):
    raise NotImplementedError("write your pallas kernel here")


# speedup vs baseline: 53.6724x; 53.6724x over previous
import numpy as np
import jax
import jax.numpy as jnp
from jax.experimental import pallas as pl

IN_DIM = 128
HIDDEN_DIM = 64
OUT_DIM = 128
HEADS_SPACE = 2
K = 9
B = 64
H = 28
W = 28
P = H * W


def _build_static_graph():
    ii, jj = np.meshgrid(np.arange(H), np.arange(W), indexing='ij')
    coords = np.stack([ii.ravel(), jj.ravel()], axis=-1).astype(np.float32)
    coords = coords / coords.max()
    d2 = ((coords[:, None, :] - coords[None, :, :]) ** 2).sum(-1)
    np.fill_diagonal(d2, np.inf)
    nbr = np.argsort(d2, axis=1)[:, :K]
    offs = nbr - np.arange(P)[:, None]
    uniq = np.unique(offs)
    mask = np.zeros((P, len(uniq)), np.float32)
    for j, d in enumerate(uniq):
        mask[:, j] = (offs == d).any(axis=1)
    assert mask.sum() == P * K
    return [int(d) for d in uniq], mask


_OFFSETS, _MASK_NP = _build_static_graph()
_NOFF = len(_OFFSETS)
_PAD = max(abs(d) for d in _OFFSETS)


def _lrelu(v):
    return jnp.where(v > 0, v, 0.2 * v)


def _gat1_body(x_ref, w1_ref, avec_ref, b1_ref, maskf_ref, xs_ref, isum_ref):
    xm = x_ref[0]
    xw = jax.lax.dot_general(xm, w1_ref[...], (((0,), (0,)), ((), ())),
                             preferred_element_type=jnp.float32)
    asd = jax.lax.dot_general(xw, avec_ref[...], (((1,), (0,)), ((), ())),
                              preferred_element_type=jnp.float32)
    a_s = asd[:, 0:2]
    a_d = asd[:, 2:4]
    zpad = jnp.zeros((_PAD, 2), jnp.float32)
    a_s_pad = jnp.concatenate([zpad, a_s, zpad], axis=0)

    e0 = _lrelu(a_s + a_d)
    m = e0
    e_list = []
    for j, d in enumerate(_OFFSETS):
        es = _lrelu(a_s_pad[_PAD + d:_PAD + d + P, :] + a_d)
        e_list.append(es)
        mk = maskf_ref[:, j:j + 1]
        m = jnp.maximum(m, jnp.where(mk > 0, es, -1e30))

    ex0 = jnp.exp(e0 - m)
    den = ex0
    num0 = ex0[:, 0:1] * xw[:, 0:HIDDEN_DIM]
    num1 = ex0[:, 1:2] * xw[:, HIDDEN_DIM:]
    zpadw = jnp.zeros((_PAD, HEADS_SPACE * HIDDEN_DIM), jnp.float32)
    xw_pad = jnp.concatenate([zpadw, xw, zpadw], axis=0)
    for j, d in enumerate(_OFFSETS):
        mk = maskf_ref[:, j:j + 1]
        ex = jnp.exp(e_list[j] - m) * mk
        den = den + ex
        sh = xw_pad[_PAD + d:_PAD + d + P, :]
        num0 = num0 + ex[:, 0:1] * sh[:, 0:HIDDEN_DIM]
        num1 = num1 + ex[:, 1:2] * sh[:, HIDDEN_DIM:]

    xs = jnp.concatenate([num0 / (den[:, 0:1] + 1e-16),
                          num1 / (den[:, 1:2] + 1e-16)], axis=1) + b1_ref[...]
    xs_ref[0] = xs
    isum_ref[0] = jnp.sum(xs, axis=0, keepdims=True)


def _gat2_body(isum_ref, bidc_ref, bidr_ref, w2_ref, as2_ref, ad2_ref,
               b2_ref, fv_ref):
    emb = isum_ref[...] * (1.0 / P)
    xw2 = jnp.dot(emb, w2_ref[...], preferred_element_type=jnp.float32)
    a_s_row = jax.lax.dot_general(as2_ref[...], xw2, (((1,), (1,)), ((), ())),
                                  preferred_element_type=jnp.float32)
    a_d_col = jnp.dot(xw2, ad2_ref[...],
                      preferred_element_type=jnp.float32)
    e = _lrelu(a_s_row + a_d_col)
    mval = bidc_ref[...] == bidr_ref[...]
    em = jnp.where(mval, e, -1e30)
    mrow = jnp.max(em, axis=1, keepdims=True)
    ex = jnp.exp(em - mrow) * mval.astype(jnp.float32)
    den = jnp.sum(ex, axis=1, keepdims=True) + 1e-16
    fv = jnp.dot(ex, xw2, preferred_element_type=jnp.float32) / den
    fv_ref[...] = fv + b2_ref[...]


def _blend_body(xs_ref, fv_ref, o_ref):
    o_ref[0] = 0.5 * xs_ref[0] + 0.5 * fv_ref[0]


def kernel(x, batch_id_all, batch_id, W1, a_src1, a_dst1, b1,
           W2, a_src2, a_dst2, b2):
    del batch_id
    f32 = jnp.float32
    x2 = x.reshape(B, IN_DIM, P)

    z = jnp.zeros((HIDDEN_DIM,), f32)
    avec = jnp.stack([
        jnp.concatenate([a_src1[0], z]),
        jnp.concatenate([z, a_src1[1]]),
        jnp.concatenate([a_dst1[0], z]),
        jnp.concatenate([z, a_dst1[1]]),
    ], axis=1)

    maskf = jnp.asarray(_MASK_NP)

    xs, isum = pl.pallas_call(
        _gat1_body,
        grid=(B,),
        in_specs=[
            pl.BlockSpec((1, IN_DIM, P), lambda i: (i, 0, 0)),
            pl.BlockSpec((IN_DIM, HEADS_SPACE * HIDDEN_DIM), lambda i: (0, 0)),
            pl.BlockSpec((IN_DIM, 4), lambda i: (0, 0)),
            pl.BlockSpec((1, HEADS_SPACE * HIDDEN_DIM), lambda i: (0, 0)),
            pl.BlockSpec((P, _NOFF), lambda i: (0, 0)),
        ],
        out_specs=[
            pl.BlockSpec((1, P, HEADS_SPACE * HIDDEN_DIM), lambda i: (i, 0, 0)),
            pl.BlockSpec((1, 1, HEADS_SPACE * HIDDEN_DIM), lambda i: (i, 0, 0)),
        ],
        out_shape=[
            jax.ShapeDtypeStruct((B, P, HEADS_SPACE * HIDDEN_DIM), f32),
            jax.ShapeDtypeStruct((B, 1, HEADS_SPACE * HIDDEN_DIM), f32),
        ],
    )(x2, W1, avec, b1.reshape(1, -1), maskf)
    isum = isum.reshape(B, HEADS_SPACE * HIDDEN_DIM)

    fv = pl.pallas_call(
        _gat2_body,
        out_shape=jax.ShapeDtypeStruct((B, OUT_DIM), f32),
    )(isum, batch_id_all.reshape(B, 1), batch_id_all.reshape(1, B),
      W2, a_src2, a_dst2.reshape(-1, 1), b2.reshape(1, -1))

    out = pl.pallas_call(
        _blend_body,
        grid=(B,),
        in_specs=[
            pl.BlockSpec((1, P, OUT_DIM), lambda i: (i, 0, 0)),
            pl.BlockSpec((1, 1, OUT_DIM), lambda i: (i, 0, 0)),
        ],
        out_specs=pl.BlockSpec((1, P, OUT_DIM), lambda i: (i, 0, 0)),
        out_shape=jax.ShapeDtypeStruct((B, P, OUT_DIM), f32),
    )(xs, fv.reshape(B, 1, OUT_DIM))

    return out.reshape(B, OUT_DIM, H, W)


# speedup vs baseline: 162.6580x; 3.0306x over previous
import numpy as np
import jax
import jax.numpy as jnp
from jax.experimental import pallas as pl

IN_DIM = 128
HIDDEN_DIM = 64
OUT_DIM = 128
HEADS_SPACE = 2
K = 9
B = 64
H = 28
W = 28
P = H * W


def _build_static_graph():
    ii, jj = np.meshgrid(np.arange(H), np.arange(W), indexing='ij')
    coords = np.stack([ii.ravel(), jj.ravel()], axis=-1).astype(np.float32)
    coords = coords / coords.max()
    d2 = ((coords[:, None, :] - coords[None, :, :]) ** 2).sum(-1)
    np.fill_diagonal(d2, np.inf)
    nbr = np.argsort(d2, axis=1)[:, :K]
    offs = nbr - np.arange(P)[:, None]
    uniq = np.unique(offs)
    nslot = -((len(uniq) + 1) // -8) * 8
    maskadd = np.full((nslot, P), -1e30, np.float32)
    for j, d in enumerate(uniq):
        maskadd[j] = np.where((offs == d).any(axis=1), 0.0, -1e30)
    maskadd[len(uniq)] = 0.0
    assert (maskadd[:len(uniq)] == 0.0).sum() == P * K
    return [int(d) for d in uniq], maskadd, nslot


_OFFSETS, _MASKADD_NP, _NSLOT = _build_static_graph()
_NOFF = len(_OFFSETS)
_PAD = max(abs(d) for d in _OFFSETS)


def _build_expand():
    nsl = _NOFF + 1
    e2 = np.zeros((2 * _NSLOT, nsl * 128), np.float32)
    for j in range(nsl):
        e2[j, j * 128:j * 128 + HIDDEN_DIM] = 1.0
        e2[_NSLOT + j, j * 128 + HIDDEN_DIM:(j + 1) * 128] = 1.0
    return e2


_EXPAND_NP = _build_expand()


def _lrelu(v):
    return jnp.where(v > 0, v, 0.2 * v)


_FRONT = 88


def _gat1_body(x_ref, w1_ref, avec_ref, b1_ref, maskadd_ref, expand_ref,
               xs_ref, isum_ref):
    f32 = jnp.float32
    xm = x_ref[0]
    w1 = w1_ref[...]
    xw = jax.lax.dot_general(xm, w1, (((0,), (0,)), ((), ())),
                             preferred_element_type=f32)
    avw = jnp.dot(w1, avec_ref[...], preferred_element_type=f32)
    aT = jax.lax.dot_general(avw, xm, (((0,), (0,)), ((), ())),
                             preferred_element_type=f32)
    a_sT = aT[0:2]
    a_dT = aT[2:4]
    zl = jnp.zeros((2, _FRONT), f32)
    a_sT_pad = jnp.concatenate([zl, a_sT, zl], axis=1)

    rows0, rows1 = [], []
    for d in _OFFSETS:
        es = _lrelu(a_sT_pad[:, _FRONT + d:_FRONT + d + P] + a_dT)
        rows0.append(es[0:1])
        rows1.append(es[1:2])
    es_self = _lrelu(a_sT + a_dT)
    rows0.append(es_self[0:1])
    rows1.append(es_self[1:2])
    zrows = jnp.zeros((_NSLOT - _NOFF - 1, P), f32)
    maskadd = maskadd_ref[...]
    exn_heads = []
    for rows in (rows0, rows1):
        est = jnp.concatenate(rows + [zrows], axis=0) + maskadd
        mrow = jnp.max(est, axis=0, keepdims=True)
        ex = jnp.exp(est - mrow)
        den = jnp.sum(ex, axis=0, keepdims=True)
        exn_heads.append(ex * (1.0 / (den + 1e-16)))
    exc = jnp.concatenate(exn_heads, axis=0).T
    bc = jax.lax.dot_general(exc, expand_ref[...], (((1,), (0,)), ((), ())),
                             preferred_element_type=f32)

    zp = jnp.zeros((_FRONT, HEADS_SPACE * HIDDEN_DIM), f32)
    xw_pad = jnp.concatenate([zp, xw, zp], axis=0)
    rots = {}
    for d in _OFFSETS:
        r = (_FRONT + d) % 8
        if r not in rots:
            rots[r] = xw_pad[r:r + P + 2 * _FRONT - 8]
    num = bc[:, _NOFF * 128:(_NOFF + 1) * 128] * xw
    for j, d in enumerate(_OFFSETS):
        s = _FRONT + d
        r = s % 8
        sh = rots[r][s - r:s - r + P]
        num = num + bc[:, j * 128:(j + 1) * 128] * sh

    xs = num + b1_ref[...]
    xs_ref[0] = xs
    isum_ref[0] = jnp.sum(xs, axis=0, keepdims=True)


def _gat2_body(isum_ref, bidc_ref, bidr_ref, w2_ref, as2_ref, ad2_ref,
               b2_ref, fv_ref):
    emb = isum_ref[...] * (1.0 / P)
    xw2 = jnp.dot(emb, w2_ref[...], preferred_element_type=jnp.float32)
    a_s_row = jax.lax.dot_general(as2_ref[...], xw2, (((1,), (1,)), ((), ())),
                                  preferred_element_type=jnp.float32)
    a_d_col = jnp.dot(xw2, ad2_ref[...],
                      preferred_element_type=jnp.float32)
    e = _lrelu(a_s_row + a_d_col)
    mval = bidc_ref[...] == bidr_ref[...]
    em = jnp.where(mval, e, -1e30)
    mrow = jnp.max(em, axis=1, keepdims=True)
    ex = jnp.exp(em - mrow) * mval.astype(jnp.float32)
    den = jnp.sum(ex, axis=1, keepdims=True) + 1e-16
    fv = jnp.dot(ex, xw2, preferred_element_type=jnp.float32) / den
    fv_ref[...] = fv + b2_ref[...]


def _blend_body(xs_ref, fv_ref, o_ref):
    o_ref[0] = 0.5 * xs_ref[0] + 0.5 * fv_ref[0]


def kernel(x, batch_id_all, batch_id, W1, a_src1, a_dst1, b1,
           W2, a_src2, a_dst2, b2):
    del batch_id
    f32 = jnp.float32
    x2 = x.reshape(B, IN_DIM, P)

    z = jnp.zeros((HIDDEN_DIM,), f32)
    avec = jnp.stack([
        jnp.concatenate([a_src1[0], z]),
        jnp.concatenate([z, a_src1[1]]),
        jnp.concatenate([a_dst1[0], z]),
        jnp.concatenate([z, a_dst1[1]]),
    ], axis=1)

    maskadd = jnp.asarray(_MASKADD_NP)

    xs, isum = pl.pallas_call(
        _gat1_body,
        grid=(B,),
        in_specs=[
            pl.BlockSpec((1, IN_DIM, P), lambda i: (i, 0, 0)),
            pl.BlockSpec((IN_DIM, HEADS_SPACE * HIDDEN_DIM), lambda i: (0, 0)),
            pl.BlockSpec((IN_DIM, 4), lambda i: (0, 0)),
            pl.BlockSpec((1, HEADS_SPACE * HIDDEN_DIM), lambda i: (0, 0)),
            pl.BlockSpec((_NSLOT, P), lambda i: (0, 0)),
            pl.BlockSpec(_EXPAND_NP.shape, lambda i: (0, 0)),
        ],
        out_specs=[
            pl.BlockSpec((1, P, HEADS_SPACE * HIDDEN_DIM), lambda i: (i, 0, 0)),
            pl.BlockSpec((1, 1, HEADS_SPACE * HIDDEN_DIM), lambda i: (i, 0, 0)),
        ],
        out_shape=[
            jax.ShapeDtypeStruct((B, P, HEADS_SPACE * HIDDEN_DIM), f32),
            jax.ShapeDtypeStruct((B, 1, HEADS_SPACE * HIDDEN_DIM), f32),
        ],
    )(x2, W1, avec, b1.reshape(1, -1), maskadd, jnp.asarray(_EXPAND_NP))
    isum = isum.reshape(B, HEADS_SPACE * HIDDEN_DIM)

    fv = pl.pallas_call(
        _gat2_body,
        out_shape=jax.ShapeDtypeStruct((B, OUT_DIM), f32),
    )(isum, batch_id_all.reshape(B, 1), batch_id_all.reshape(1, B),
      W2, a_src2, a_dst2.reshape(-1, 1), b2.reshape(1, -1))

    out = pl.pallas_call(
        _blend_body,
        grid=(B,),
        in_specs=[
            pl.BlockSpec((1, P, OUT_DIM), lambda i: (i, 0, 0)),
            pl.BlockSpec((1, 1, OUT_DIM), lambda i: (i, 0, 0)),
        ],
        out_specs=pl.BlockSpec((1, P, OUT_DIM), lambda i: (i, 0, 0)),
        out_shape=jax.ShapeDtypeStruct((B, P, OUT_DIM), f32),
    )(xs, fv.reshape(B, 1, OUT_DIM))

    return out.reshape(B, OUT_DIM, H, W)


# speedup vs baseline: 182.9066x; 1.1245x over previous
import numpy as np
import jax
import jax.numpy as jnp
from jax.experimental import pallas as pl

IN_DIM = 128
HIDDEN_DIM = 64
OUT_DIM = 128
HEADS_SPACE = 2
K = 9
B = 64
H = 28
W = 28
P = H * W


def _build_static_graph():
    ii, jj = np.meshgrid(np.arange(H), np.arange(W), indexing='ij')
    coords = np.stack([ii.ravel(), jj.ravel()], axis=-1).astype(np.float32)
    coords = coords / coords.max()
    d2 = ((coords[:, None, :] - coords[None, :, :]) ** 2).sum(-1)
    np.fill_diagonal(d2, np.inf)
    nbr = np.argsort(d2, axis=1)[:, :K]
    offs = nbr - np.arange(P)[:, None]
    uniq = np.unique(offs)
    nslot = -((len(uniq) + 1) // -8) * 8
    maskadd = np.full((nslot, P), -1e30, np.float32)
    for j, d in enumerate(uniq):
        maskadd[j] = np.where((offs == d).any(axis=1), 0.0, -1e30)
    maskadd[len(uniq)] = 0.0
    assert (maskadd[:len(uniq)] == 0.0).sum() == P * K
    return [int(d) for d in uniq], maskadd, nslot


_OFFSETS, _MASKADD_NP, _NSLOT = _build_static_graph()
_NOFF = len(_OFFSETS)
_PAD = max(abs(d) for d in _OFFSETS)


def _build_expand():
    nsl = _NOFF + 1
    e2 = np.zeros((2 * _NSLOT, nsl * 128), np.float32)
    for j in range(nsl):
        e2[j, j * 128:j * 128 + HIDDEN_DIM] = 1.0
        e2[_NSLOT + j, j * 128 + HIDDEN_DIM:(j + 1) * 128] = 1.0
    return e2


_EXPAND_NP = _build_expand()


def _lrelu(v):
    return jnp.where(v > 0, v, 0.2 * v)


_FRONT = 88


def _gat1_body(x_ref, w1_ref, avec_ref, b1_ref, maskadd_ref, expand_ref,
               xs_ref, isum_ref):
    f32 = jnp.float32
    xm = x_ref[0]
    w1 = w1_ref[...]
    xw = jax.lax.dot_general(xm, w1, (((0,), (0,)), ((), ())),
                             preferred_element_type=f32)
    avw = jnp.dot(w1, avec_ref[...], preferred_element_type=f32)
    aT = jax.lax.dot_general(avw, xm, (((0,), (0,)), ((), ())),
                             preferred_element_type=f32)
    a_sT = aT[0:2]
    a_dT = aT[2:4]
    zl = jnp.zeros((2, _FRONT), f32)
    a_sT_pad = jnp.concatenate([zl, a_sT, zl], axis=1)

    rows0, rows1 = [], []
    for d in _OFFSETS:
        es = _lrelu(a_sT_pad[:, _FRONT + d:_FRONT + d + P] + a_dT)
        rows0.append(es[0:1])
        rows1.append(es[1:2])
    es_self = _lrelu(a_sT + a_dT)
    rows0.append(es_self[0:1])
    rows1.append(es_self[1:2])
    zrows = jnp.zeros((_NSLOT - _NOFF - 1, P), f32)
    maskadd = maskadd_ref[...]
    exn_heads = []
    for rows in (rows0, rows1):
        est = jnp.concatenate(rows + [zrows], axis=0) + maskadd
        mrow = jnp.max(est, axis=0, keepdims=True)
        ex = jnp.exp(est - mrow)
        den = jnp.sum(ex, axis=0, keepdims=True)
        exn_heads.append(ex * (1.0 / (den + 1e-16)))
    exc = jnp.concatenate(exn_heads, axis=0).T
    expand = expand_ref[...]

    zp = jnp.zeros((_FRONT, HEADS_SPACE * HIDDEN_DIM), f32)
    xw_pad = jnp.concatenate([zp, xw, zp], axis=0)
    rots = {}
    for d in _OFFSETS:
        r = (_FRONT + d) % 8
        if r not in rots:
            rots[r] = xw_pad[r:r + P + 2 * _FRONT - 8]
    shifts = []
    for d in _OFFSETS:
        s = _FRONT + d
        r = s % 8
        shifts.append(rots[r][s - r:s - r + P])
    shifts.append(xw)
    num = None
    csz = 7
    for c0 in range(0, _NOFF + 1, csz):
        c1 = min(c0 + csz, _NOFF + 1)
        bc = jax.lax.dot_general(
            exc, expand[:, c0 * 128:c1 * 128], (((1,), (0,)), ((), ())),
            preferred_element_type=f32)
        for j in range(c0, c1):
            t = bc[:, (j - c0) * 128:(j - c0 + 1) * 128] * shifts[j]
            num = t if num is None else num + t

    xs = num + b1_ref[...]
    xs_ref[0] = xs
    isum_ref[0] = jnp.sum(xs, axis=0, keepdims=True)


def _gat2_body(isum_ref, bidc_ref, bidr_ref, w2_ref, as2_ref, ad2_ref,
               b2_ref, fv_ref):
    emb = isum_ref[...] * (1.0 / P)
    xw2 = jnp.dot(emb, w2_ref[...], preferred_element_type=jnp.float32)
    a_s_row = jax.lax.dot_general(as2_ref[...], xw2, (((1,), (1,)), ((), ())),
                                  preferred_element_type=jnp.float32)
    a_d_col = jnp.dot(xw2, ad2_ref[...],
                      preferred_element_type=jnp.float32)
    e = _lrelu(a_s_row + a_d_col)
    mval = bidc_ref[...] == bidr_ref[...]
    em = jnp.where(mval, e, -1e30)
    mrow = jnp.max(em, axis=1, keepdims=True)
    ex = jnp.exp(em - mrow) * mval.astype(jnp.float32)
    den = jnp.sum(ex, axis=1, keepdims=True) + 1e-16
    fv = jnp.dot(ex, xw2, preferred_element_type=jnp.float32) / den
    fv_ref[...] = fv + b2_ref[...]


def _blend_body(xs_ref, fv_ref, o_ref):
    o_ref[0] = 0.5 * xs_ref[0] + 0.5 * fv_ref[0]


def kernel(x, batch_id_all, batch_id, W1, a_src1, a_dst1, b1,
           W2, a_src2, a_dst2, b2):
    del batch_id
    f32 = jnp.float32
    x2 = x.reshape(B, IN_DIM, P)

    z = jnp.zeros((HIDDEN_DIM,), f32)
    avec = jnp.stack([
        jnp.concatenate([a_src1[0], z]),
        jnp.concatenate([z, a_src1[1]]),
        jnp.concatenate([a_dst1[0], z]),
        jnp.concatenate([z, a_dst1[1]]),
    ], axis=1)

    maskadd = jnp.asarray(_MASKADD_NP)

    xs, isum = pl.pallas_call(
        _gat1_body,
        grid=(B,),
        in_specs=[
            pl.BlockSpec((1, IN_DIM, P), lambda i: (i, 0, 0)),
            pl.BlockSpec((IN_DIM, HEADS_SPACE * HIDDEN_DIM), lambda i: (0, 0)),
            pl.BlockSpec((IN_DIM, 4), lambda i: (0, 0)),
            pl.BlockSpec((1, HEADS_SPACE * HIDDEN_DIM), lambda i: (0, 0)),
            pl.BlockSpec((_NSLOT, P), lambda i: (0, 0)),
            pl.BlockSpec(_EXPAND_NP.shape, lambda i: (0, 0)),
        ],
        out_specs=[
            pl.BlockSpec((1, P, HEADS_SPACE * HIDDEN_DIM), lambda i: (i, 0, 0)),
            pl.BlockSpec((1, 1, HEADS_SPACE * HIDDEN_DIM), lambda i: (i, 0, 0)),
        ],
        out_shape=[
            jax.ShapeDtypeStruct((B, P, HEADS_SPACE * HIDDEN_DIM), f32),
            jax.ShapeDtypeStruct((B, 1, HEADS_SPACE * HIDDEN_DIM), f32),
        ],
    )(x2, W1, avec, b1.reshape(1, -1), maskadd, jnp.asarray(_EXPAND_NP))
    isum = isum.reshape(B, HEADS_SPACE * HIDDEN_DIM)

    fv = pl.pallas_call(
        _gat2_body,
        out_shape=jax.ShapeDtypeStruct((B, OUT_DIM), f32),
    )(isum, batch_id_all.reshape(B, 1), batch_id_all.reshape(1, B),
      W2, a_src2, a_dst2.reshape(-1, 1), b2.reshape(1, -1))

    out = pl.pallas_call(
        _blend_body,
        grid=(B,),
        in_specs=[
            pl.BlockSpec((1, P, OUT_DIM), lambda i: (i, 0, 0)),
            pl.BlockSpec((1, 1, OUT_DIM), lambda i: (i, 0, 0)),
        ],
        out_specs=pl.BlockSpec((1, P, OUT_DIM), lambda i: (i, 0, 0)),
        out_shape=jax.ShapeDtypeStruct((B, P, OUT_DIM), f32),
    )(xs, fv.reshape(B, 1, OUT_DIM))

    return out.reshape(B, OUT_DIM, H, W)


# speedup vs baseline: 187.5841x; 1.0256x over previous
import numpy as np
import jax
import jax.numpy as jnp
from jax.experimental import pallas as pl

IN_DIM = 128
HIDDEN_DIM = 64
OUT_DIM = 128
HEADS_SPACE = 2
K = 9
B = 64
H = 28
W = 28
P = H * W


def _build_static_graph():
    ii, jj = np.meshgrid(np.arange(H), np.arange(W), indexing='ij')
    coords = np.stack([ii.ravel(), jj.ravel()], axis=-1).astype(np.float32)
    coords = coords / coords.max()
    d2 = ((coords[:, None, :] - coords[None, :, :]) ** 2).sum(-1)
    np.fill_diagonal(d2, np.inf)
    nbr = np.argsort(d2, axis=1)[:, :K]
    offs = nbr - np.arange(P)[:, None]
    uniq = np.unique(offs)
    nslot = -((len(uniq) + 1) // -8) * 8
    maskadd = np.full((nslot, P), -1e30, np.float32)
    for j, d in enumerate(uniq):
        maskadd[j] = np.where((offs == d).any(axis=1), 0.0, -1e30)
    maskadd[len(uniq)] = 0.0
    assert (maskadd[:len(uniq)] == 0.0).sum() == P * K
    return [int(d) for d in uniq], maskadd, nslot


_OFFSETS, _MASKADD_NP, _NSLOT = _build_static_graph()
_NOFF = len(_OFFSETS)
_PAD = max(abs(d) for d in _OFFSETS)


def _build_expand():
    nsl = _NOFF + 1
    e2 = np.zeros((2 * _NSLOT, nsl * 128), np.float32)
    for j in range(nsl):
        e2[j, j * 128:j * 128 + HIDDEN_DIM] = 1.0
        e2[_NSLOT + j, j * 128 + HIDDEN_DIM:(j + 1) * 128] = 1.0
    return e2


_EXPAND_NP = _build_expand()


def _lrelu(v):
    return jnp.where(v > 0, v, 0.2 * v)


_FRONT = 88


_G = 4


def _gat1_body(x_ref, w1_ref, avec_ref, b1_ref, maskadd_ref, expand_ref,
               xs_ref, isum_ref):
    for g in range(_G):
        _gat1_one(g, x_ref, w1_ref, avec_ref, b1_ref, maskadd_ref,
                  expand_ref, xs_ref, isum_ref)


def _gat1_one(g, x_ref, w1_ref, avec_ref, b1_ref, maskadd_ref, expand_ref,
              xs_ref, isum_ref):
    f32 = jnp.float32
    xm = x_ref[g]
    w1 = w1_ref[...]
    xw = jax.lax.dot_general(xm, w1, (((0,), (0,)), ((), ())),
                             preferred_element_type=f32)
    avw = jnp.dot(w1, avec_ref[...], preferred_element_type=f32)
    aT = jax.lax.dot_general(avw, xm, (((0,), (0,)), ((), ())),
                             preferred_element_type=f32)
    a_sT = aT[0:2]
    a_dT = aT[2:4]
    zl = jnp.zeros((2, _FRONT), f32)
    a_sT_pad = jnp.concatenate([zl, a_sT, zl], axis=1)

    rows0, rows1 = [], []
    for d in _OFFSETS:
        es = _lrelu(a_sT_pad[:, _FRONT + d:_FRONT + d + P] + a_dT)
        rows0.append(es[0:1])
        rows1.append(es[1:2])
    es_self = _lrelu(a_sT + a_dT)
    rows0.append(es_self[0:1])
    rows1.append(es_self[1:2])
    zrows = jnp.zeros((_NSLOT - _NOFF - 1, P), f32)
    maskadd = maskadd_ref[...]
    exn_heads = []
    for rows in (rows0, rows1):
        est = jnp.concatenate(rows + [zrows], axis=0) + maskadd
        mrow = jnp.max(est, axis=0, keepdims=True)
        ex = jnp.exp(est - mrow)
        den = jnp.sum(ex, axis=0, keepdims=True)
        exn_heads.append(ex * (1.0 / (den + 1e-16)))
    exc = jnp.concatenate(exn_heads, axis=0).T
    expand = expand_ref[...]

    zp = jnp.zeros((_FRONT, HEADS_SPACE * HIDDEN_DIM), f32)
    xw_pad = jnp.concatenate([zp, xw, zp], axis=0)
    rots = {}
    for d in _OFFSETS:
        r = (_FRONT + d) % 8
        if r not in rots:
            rots[r] = xw_pad[r:r + P + 2 * _FRONT - 8]
    shifts = []
    for d in _OFFSETS:
        s = _FRONT + d
        r = s % 8
        shifts.append(rots[r][s - r:s - r + P])
    shifts.append(xw)
    num = None
    csz = 7
    for c0 in range(0, _NOFF + 1, csz):
        c1 = min(c0 + csz, _NOFF + 1)
        bc = jax.lax.dot_general(
            exc, expand[:, c0 * 128:c1 * 128], (((1,), (0,)), ((), ())),
            preferred_element_type=f32)
        for j in range(c0, c1):
            t = bc[:, (j - c0) * 128:(j - c0 + 1) * 128] * shifts[j]
            num = t if num is None else num + t

    xs = num + b1_ref[...]
    xs_ref[g] = xs
    isum_ref[g] = jnp.sum(xs, axis=0, keepdims=True)


def _gat2_body(isum_ref, bidc_ref, bidr_ref, w2_ref, as2_ref, ad2_ref,
               b2_ref, fv_ref):
    emb = isum_ref[...] * (1.0 / P)
    xw2 = jnp.dot(emb, w2_ref[...], preferred_element_type=jnp.float32)
    a_s_row = jax.lax.dot_general(as2_ref[...], xw2, (((1,), (1,)), ((), ())),
                                  preferred_element_type=jnp.float32)
    a_d_col = jnp.dot(xw2, ad2_ref[...],
                      preferred_element_type=jnp.float32)
    e = _lrelu(a_s_row + a_d_col)
    mval = bidc_ref[...] == bidr_ref[...]
    em = jnp.where(mval, e, -1e30)
    mrow = jnp.max(em, axis=1, keepdims=True)
    ex = jnp.exp(em - mrow) * mval.astype(jnp.float32)
    den = jnp.sum(ex, axis=1, keepdims=True) + 1e-16
    fv = jnp.dot(ex, xw2, preferred_element_type=jnp.float32) / den
    fv_ref[...] = fv + b2_ref[...]


def _blend_body(xs_ref, fv_ref, o_ref):
    o_ref[0] = 0.5 * xs_ref[0] + 0.5 * fv_ref[0]


def kernel(x, batch_id_all, batch_id, W1, a_src1, a_dst1, b1,
           W2, a_src2, a_dst2, b2):
    del batch_id
    f32 = jnp.float32
    x2 = x.reshape(B, IN_DIM, P)

    z = jnp.zeros((HIDDEN_DIM,), f32)
    avec = jnp.stack([
        jnp.concatenate([a_src1[0], z]),
        jnp.concatenate([z, a_src1[1]]),
        jnp.concatenate([a_dst1[0], z]),
        jnp.concatenate([z, a_dst1[1]]),
    ], axis=1)

    maskadd = jnp.asarray(_MASKADD_NP)

    xs, isum = pl.pallas_call(
        _gat1_body,
        grid=(B // _G,),
        in_specs=[
            pl.BlockSpec((_G, IN_DIM, P), lambda i: (i, 0, 0)),
            pl.BlockSpec((IN_DIM, HEADS_SPACE * HIDDEN_DIM), lambda i: (0, 0)),
            pl.BlockSpec((IN_DIM, 4), lambda i: (0, 0)),
            pl.BlockSpec((1, HEADS_SPACE * HIDDEN_DIM), lambda i: (0, 0)),
            pl.BlockSpec((_NSLOT, P), lambda i: (0, 0)),
            pl.BlockSpec(_EXPAND_NP.shape, lambda i: (0, 0)),
        ],
        out_specs=[
            pl.BlockSpec((_G, P, HEADS_SPACE * HIDDEN_DIM), lambda i: (i, 0, 0)),
            pl.BlockSpec((_G, 1, HEADS_SPACE * HIDDEN_DIM), lambda i: (i, 0, 0)),
        ],
        out_shape=[
            jax.ShapeDtypeStruct((B, P, HEADS_SPACE * HIDDEN_DIM), f32),
            jax.ShapeDtypeStruct((B, 1, HEADS_SPACE * HIDDEN_DIM), f32),
        ],
    )(x2, W1, avec, b1.reshape(1, -1), maskadd, jnp.asarray(_EXPAND_NP))
    isum = isum.reshape(B, HEADS_SPACE * HIDDEN_DIM)

    fv = pl.pallas_call(
        _gat2_body,
        out_shape=jax.ShapeDtypeStruct((B, OUT_DIM), f32),
    )(isum, batch_id_all.reshape(B, 1), batch_id_all.reshape(1, B),
      W2, a_src2, a_dst2.reshape(-1, 1), b2.reshape(1, -1))

    out = pl.pallas_call(
        _blend_body,
        grid=(B,),
        in_specs=[
            pl.BlockSpec((1, P, OUT_DIM), lambda i: (i, 0, 0)),
            pl.BlockSpec((1, 1, OUT_DIM), lambda i: (i, 0, 0)),
        ],
        out_specs=pl.BlockSpec((1, P, OUT_DIM), lambda i: (i, 0, 0)),
        out_shape=jax.ShapeDtypeStruct((B, P, OUT_DIM), f32),
    )(xs, fv.reshape(B, 1, OUT_DIM))

    return out.reshape(B, OUT_DIM, H, W)
